# Initial kernel scaffold; baseline (speedup 1.0000x reference)
import jax
import jax.numpy as jnp
from jax.experimental import pallas as pl


def kernel(x, W_enc, b_enc, k):
    raise NotImplementedError("write your pallas kernel here")


# speedup vs baseline: 2.2973x; 2.2973x over previous
import numpy as np
import jax
import jax.numpy as jnp
from jax import lax
from jax.experimental import pallas as pl
from jax.experimental.pallas import tpu as pltpu

_B = 32
_D = 2048
_NF = 32768
_MAXK = 64
_TN = 2048
_NT = _NF // _TN
_CH = 4096
_NCH = _NF // _CH

def _topk_mask_body(x_ref, w_ref, b_ref, k_ref, out_ref, skey_ref):
    _MININT = jnp.int32(-2147483648)
    _LOW31 = jnp.int32(2147483647)
    i = pl.program_id(0)
    z = jnp.dot(x_ref[...], w_ref[...], preferred_element_type=jnp.float32)
    z = z + b_ref[...]
    out_ref[:, pl.ds(i * _TN, _TN)] = z

    @pl.when(i == _NT - 1)
    def _finalize():
        kk = jnp.minimum(k_ref[0], _MAXK).astype(jnp.int32)
        zeros = jnp.zeros((_B, 1), jnp.int32)

        def _build(c, _):
            zc = out_ref[:, pl.ds(c * _CH, _CH)]
            bits = lax.bitcast_convert_type(zc, jnp.int32)
            skey_ref[:, pl.ds(c * _CH, _CH)] = bits ^ (
                lax.shift_right_arithmetic(bits, 31) & _LOW31
            )
            return 0

        lax.fori_loop(0, _NCH, _build, 0)

        def _count_ge(ts):
            def cb(c, cnt):
                s = skey_ref[:, pl.ds(c * _CH, _CH)]
                return cnt + jnp.sum((s >= ts).astype(jnp.int32), axis=1,
                                     keepdims=True)
            return lax.fori_loop(0, _NCH, cb, zeros)

        def _bit(j, t_u):
            bitv = lax.shift_left(jnp.int32(1), 31 - j)
            cand_u = t_u | bitv
            cnt = _count_ge(cand_u ^ _MININT)
            return jnp.where(cnt >= kk, cand_u, t_u)

        t_u = lax.fori_loop(0, 32, _bit, zeros)
        t_s = t_u ^ _MININT

        def _gcounts(c, carry):
            ge, gt = carry
            s = skey_ref[:, pl.ds(c * _CH, _CH)]
            ge = ge + jnp.sum((s >= t_s).astype(jnp.int32), axis=1,
                              keepdims=True)
            gt = gt + jnp.sum((s > t_s).astype(jnp.int32), axis=1,
                              keepdims=True)
            return ge, gt

        cnt_ge, cnt_gt = lax.fori_loop(0, _NCH, _gcounts, (zeros, zeros))
        r = kk - cnt_gt

        def _cols(c):
            return c * _CH + lax.broadcasted_iota(jnp.int32, (_B, _CH), 1)

        def _idx_search():
            def cnt_eq_le(m):
                def cb(c, cnt):
                    s = skey_ref[:, pl.ds(c * _CH, _CH)]
                    hit = (s == t_s) & (_cols(c) <= m)
                    return cnt + jnp.sum(hit.astype(jnp.int32), axis=1,
                                         keepdims=True)
                return lax.fori_loop(0, _NCH, cb, zeros)

            def bs(j, lohi):
                lo, hi = lohi
                mid = lax.shift_right_arithmetic(lo + hi, 1)
                ok = cnt_eq_le(mid) >= r
                return (jnp.where(ok, lo, mid + 1), jnp.where(ok, mid, hi))

            lo, hi = lax.fori_loop(
                0, 15, bs, (zeros, jnp.full((_B, 1), _NF - 1, jnp.int32)))
            return hi

        need = jnp.any(cnt_ge > kk)
        c_idx = lax.cond(need, _idx_search,
                         lambda: jnp.full((_B, 1), _NF - 1, jnp.int32))

        def _writeback(c, _):
            s = skey_ref[:, pl.ds(c * _CH, _CH)]
            zc = out_ref[:, pl.ds(c * _CH, _CH)]
            keep = (s > t_s) | ((s == t_s) & (_cols(c) <= c_idx))
            out_ref[:, pl.ds(c * _CH, _CH)] = jnp.where(
                keep, jnp.maximum(zc, 0.0), 0.0)
            return 0

        lax.fori_loop(0, _NCH, _writeback, 0)


def kernel(x, W_enc, b_enc, k):
    b2 = jnp.reshape(b_enc.astype(jnp.float32), (1, _NF))
    karr = jnp.reshape(jnp.asarray(k, jnp.int32), (1,))
    return pl.pallas_call(
        _topk_mask_body,
        grid=(_NT,),
        in_specs=[
            pl.BlockSpec((_B, _D), lambda i: (0, 0)),
            pl.BlockSpec((_D, _TN), lambda i: (0, i)),
            pl.BlockSpec((1, _TN), lambda i: (0, i)),
            pl.BlockSpec(memory_space=pltpu.SMEM),
        ],
        out_specs=pl.BlockSpec((_B, _NF), lambda i: (0, 0)),
        out_shape=jax.ShapeDtypeStruct((_B, _NF), jnp.float32),
        scratch_shapes=[pltpu.VMEM((_B, _NF), jnp.int32)],
        compiler_params=pltpu.CompilerParams(
            dimension_semantics=("arbitrary",)),
    )(x.astype(jnp.float32), W_enc.astype(jnp.float32), b2, karr)


# speedup vs baseline: 2.5142x; 1.0944x over previous
import numpy as np
import jax
import jax.numpy as jnp
from jax import lax
from jax.experimental import pallas as pl
from jax.experimental.pallas import tpu as pltpu

_B = 32
_D = 2048
_NF = 32768
_MAXK = 64
_TN = 1024
_NT = _NF // _TN
_CH = 4096
_NCH = _NF // _CH


def _topk_mask_body(x_ref, w_ref, b_ref, k_ref, out_ref, skey_ref):
    _MININT = jnp.int32(-2147483648)
    _LOW31 = jnp.int32(2147483647)
    i = pl.program_id(0)
    z = jnp.dot(x_ref[...], w_ref[...], preferred_element_type=jnp.float32)
    z = z + b_ref[...]
    out_ref[:, pl.ds(i * _TN, _TN)] = z
    bits = lax.bitcast_convert_type(z, jnp.int32)
    skey_ref[:, pl.ds(i * _TN, _TN)] = bits ^ (
        lax.shift_right_arithmetic(bits, 31) & _LOW31)

    @pl.when(i == _NT - 1)
    def _finalize():
        kk = jnp.minimum(k_ref[0], _MAXK).astype(jnp.int32)
        zeros = jnp.zeros((_B, 1), jnp.int32)

        def _count_ge(ts):
            def cb(c, cnt):
                s = skey_ref[:, pl.ds(c * _CH, _CH)]
                return cnt + jnp.sum((s >= ts).astype(jnp.int32), axis=1,
                                     keepdims=True)
            return lax.fori_loop(0, _NCH, cb, zeros)

        def _bit(j, t_u):
            bitv = lax.shift_left(jnp.int32(1), 31 - j)
            cand_u = t_u | bitv
            cnt = _count_ge(cand_u ^ _MININT)
            return jnp.where(cnt >= kk, cand_u, t_u)

        t_u24 = lax.fori_loop(0, 24, _bit, zeros)

        def _gcounts(ts):
            def cb(c, carry):
                ge, gt = carry
                s = skey_ref[:, pl.ds(c * _CH, _CH)]
                ge = ge + jnp.sum((s >= ts).astype(jnp.int32), axis=1,
                                  keepdims=True)
                gt = gt + jnp.sum((s > ts).astype(jnp.int32), axis=1,
                                  keepdims=True)
                return ge, gt
            return lax.fori_loop(0, _NCH, cb, (zeros, zeros))

        cnt_ge24, _ = _gcounts(t_u24 ^ _MININT)

        def _cols(c):
            return c * _CH + lax.broadcasted_iota(jnp.int32, (_B, _CH), 1)

        def _slow():
            t_u = lax.fori_loop(24, 32, _bit, t_u24)
            t_s = t_u ^ _MININT
            _, cnt_gt = _gcounts(t_s)
            r = kk - cnt_gt

            def cnt_eq_le(m):
                def cb(c, cnt):
                    s = skey_ref[:, pl.ds(c * _CH, _CH)]
                    hit = (s == t_s) & (_cols(c) <= m)
                    return cnt + jnp.sum(hit.astype(jnp.int32), axis=1,
                                         keepdims=True)
                return lax.fori_loop(0, _NCH, cb, zeros)

            def bs(j, lohi):
                lo, hi = lohi
                mid = lax.shift_right_arithmetic(lo + hi, 1)
                ok = cnt_eq_le(mid) >= r
                return (jnp.where(ok, lo, mid + 1), jnp.where(ok, mid, hi))

            _, hi = lax.fori_loop(
                0, 15, bs, (zeros, jnp.full((_B, 1), _NF - 1, jnp.int32)))
            return t_s, hi

        fast = jnp.all(cnt_ge24 == kk)
        t_s, c_idx = lax.cond(
            fast,
            lambda: (t_u24 ^ _MININT, jnp.full((_B, 1), _NF - 1, jnp.int32)),
            _slow)

        def _writeback(c, _):
            s = skey_ref[:, pl.ds(c * _CH, _CH)]
            zc = out_ref[:, pl.ds(c * _CH, _CH)]
            keep = (s > t_s) | ((s == t_s) & (_cols(c) <= c_idx))
            out_ref[:, pl.ds(c * _CH, _CH)] = jnp.where(
                keep, jnp.maximum(zc, 0.0), 0.0)
            return 0

        lax.fori_loop(0, _NCH, _writeback, 0)


def kernel(x, W_enc, b_enc, k):
    b2 = jnp.reshape(b_enc.astype(jnp.float32), (1, _NF))
    karr = jnp.reshape(jnp.asarray(k, jnp.int32), (1,))
    return pl.pallas_call(
        _topk_mask_body,
        grid=(_NT,),
        in_specs=[
            pl.BlockSpec((_B, _D), lambda i: (0, 0)),
            pl.BlockSpec((_D, _TN), lambda i: (0, i)),
            pl.BlockSpec((1, _TN), lambda i: (0, i)),
            pl.BlockSpec(memory_space=pltpu.SMEM),
        ],
        out_specs=pl.BlockSpec((_B, _NF), lambda i: (0, 0)),
        out_shape=jax.ShapeDtypeStruct((_B, _NF), jnp.float32),
        scratch_shapes=[pltpu.VMEM((_B, _NF), jnp.int32)],
        compiler_params=pltpu.CompilerParams(
            dimension_semantics=("arbitrary",)),
    )(x.astype(jnp.float32), W_enc.astype(jnp.float32), b2, karr)
